# Initial kernel scaffold; baseline (speedup 1.0000x reference)
import jax
import jax.numpy as jnp
from jax.experimental import pallas as pl


def kernel(features, edge_index, W1, W2, att1_src, att1_dst):
    raise NotImplementedError("write your pallas kernel here")


# speedup vs baseline: 1.7186x; 1.7186x over previous
import functools
import jax
import jax.numpy as jnp
from jax.experimental import pallas as pl
from jax.experimental.pallas import tpu as pltpu

N = 10000
E = 320000
IN_DIM = 128
HID = 64
OUT = 32

ROW_BLK = 2000


def _stage1_body(x_ref, w1_ref, as_ref, ad_ref, xp_ref, a_ref):
    xp = jnp.dot(x_ref[...], w1_ref[...], preferred_element_type=jnp.float32)
    xp_ref[...] = xp
    a_s = xp @ as_ref[...].T
    a_d = xp @ ad_ref[...].T
    a_ref[...] = jnp.concatenate([a_s, a_d], axis=1)


def _stage1(features, W1, att_s, att_d):
    grid = (N // ROW_BLK,)
    xp, a = pl.pallas_call(
        _stage1_body,
        grid=grid,
        in_specs=[
            pl.BlockSpec((ROW_BLK, IN_DIM), lambda i: (i, 0)),
            pl.BlockSpec((IN_DIM, HID), lambda i: (0, 0)),
            pl.BlockSpec((1, HID), lambda i: (0, 0)),
            pl.BlockSpec((1, HID), lambda i: (0, 0)),
        ],
        out_specs=[
            pl.BlockSpec((ROW_BLK, HID), lambda i: (i, 0)),
            pl.BlockSpec((ROW_BLK, 2), lambda i: (i, 0)),
        ],
        out_shape=[
            jax.ShapeDtypeStruct((N, HID), jnp.float32),
            jax.ShapeDtypeStruct((N, 2), jnp.float32),
        ],
    )(features, W1, att_s.reshape(1, HID), att_d.reshape(1, HID))
    return xp, a[:, 0], a[:, 1]


def _stage2_body(acc_ref, den_ref, w2_ref, h2_ref, xp3_ref):
    den = den_ref[...]
    out1 = acc_ref[...] / (den + 1e-16)
    h1 = jnp.where(out1 > 0, out1, jnp.exp(out1) - 1.0)
    w2 = w2_ref[...]
    h2 = jnp.dot(h1, w2, preferred_element_type=jnp.float32)
    h2_ref[...] = h2
    xp3_ref[...] = jnp.dot(h2, w2.T, preferred_element_type=jnp.float32)


def _stage2(acc1, den, W2):
    grid = (N // ROW_BLK,)
    h2, xp3 = pl.pallas_call(
        _stage2_body,
        grid=grid,
        in_specs=[
            pl.BlockSpec((ROW_BLK, HID), lambda i: (i, 0)),
            pl.BlockSpec((ROW_BLK, 1), lambda i: (i, 0)),
            pl.BlockSpec((HID, OUT), lambda i: (0, 0)),
        ],
        out_specs=[
            pl.BlockSpec((ROW_BLK, OUT), lambda i: (i, 0)),
            pl.BlockSpec((ROW_BLK, HID), lambda i: (i, 0)),
        ],
        out_shape=[
            jax.ShapeDtypeStruct((N, OUT), jnp.float32),
            jax.ShapeDtypeStruct((N, HID), jnp.float32),
        ],
    )(acc1, den.reshape(N, 1), W2)
    return h2, xp3


def _stage3_body(acc_ref, den_ref, w1_ref, h4_ref):
    den = den_ref[...]
    out3 = acc_ref[...] / (den + 1e-16)
    h3 = jnp.where(out3 > 0, out3, jnp.exp(out3) - 1.0)
    h4_ref[...] = jnp.dot(h3, w1_ref[...].T, preferred_element_type=jnp.float32)


def _stage3(acc3, den, W1):
    grid = (N // ROW_BLK,)
    h4 = pl.pallas_call(
        _stage3_body,
        grid=grid,
        in_specs=[
            pl.BlockSpec((ROW_BLK, HID), lambda i: (i, 0)),
            pl.BlockSpec((ROW_BLK, 1), lambda i: (i, 0)),
            pl.BlockSpec((IN_DIM, HID), lambda i: (0, 0)),
        ],
        out_specs=pl.BlockSpec((ROW_BLK, IN_DIM), lambda i: (i, 0)),
        out_shape=jax.ShapeDtypeStruct((N, IN_DIM), jnp.float32),
    )(acc3, den.reshape(N, 1), W1)
    return h4


@jax.jit
def kernel(features, edge_index, W1, W2, att1_src, att1_dst):
    src = edge_index[0]
    dst = edge_index[1]
    xp1, a_s, a_d = _stage1(features, W1, att1_src, att1_dst)

    e = jax.nn.sigmoid(a_s[src] + a_d[dst])
    w = jnp.exp(e)
    den = jax.ops.segment_sum(w, dst, num_segments=N)
    acc1 = jax.ops.segment_sum(xp1[src] * w[:, None], dst, num_segments=N)

    h2, xp3 = _stage2(acc1, den, W2)

    acc3 = jax.ops.segment_sum(xp3[src] * w[:, None], dst, num_segments=N)
    h4 = _stage3(acc3, den, W1)
    return (h2, h4)


# speedup vs baseline: 7.9336x; 4.6163x over previous
import functools
import jax
import jax.numpy as jnp
from jax import lax
from jax.experimental import pallas as pl
from jax.experimental.pallas import tpu as pltpu
from jax.experimental.pallas import tpu_sc as plsc

N = 10000
E = 320000
IN_DIM = 128
HID = 64
OUT = 32

ROW_BLK = 2000

NPAD = 10240
EPAD = 327680
NTILES = 32
EPT = EPAD // NTILES
CHUNK = 128
NCHUNK = EPT // CHUNK
ROWS_PT = NPAD // 16
GW = 128


def _stage1_body(x_ref, w1_ref, as_ref, ad_ref, xp_ref, ad_out_ref):
    xp = jnp.dot(x_ref[...], w1_ref[...], preferred_element_type=jnp.float32)
    a_s = xp @ as_ref[...].T
    a_d = xp @ ad_ref[...].T
    pad = jnp.zeros((xp.shape[0], GW - HID - 1), jnp.float32)
    xp_ref[...] = jnp.concatenate([xp, a_s, pad], axis=1)
    ad_out_ref[...] = a_d


def _stage1(features, W1, att_s, att_d):
    grid = (N // ROW_BLK,)
    xp, a_d = pl.pallas_call(
        _stage1_body,
        grid=grid,
        in_specs=[
            pl.BlockSpec((ROW_BLK, IN_DIM), lambda i: (i, 0)),
            pl.BlockSpec((IN_DIM, HID), lambda i: (0, 0)),
            pl.BlockSpec((1, HID), lambda i: (0, 0)),
            pl.BlockSpec((1, HID), lambda i: (0, 0)),
        ],
        out_specs=[
            pl.BlockSpec((ROW_BLK, GW), lambda i: (i, 0)),
            pl.BlockSpec((ROW_BLK, 1), lambda i: (i, 0)),
        ],
        out_shape=[
            jax.ShapeDtypeStruct((N, GW), jnp.float32),
            jax.ShapeDtypeStruct((N, 1), jnp.float32),
        ],
    )(features, W1, att_s.reshape(1, HID), att_d.reshape(1, HID))
    return xp, a_d[:, 0]


def _stage2_body(accp_ref, w2_ref, h2_ref, xp3_ref, den_ref):
    p = accp_ref[0] + accp_ref[1]
    den = p[:, HID:HID + 1] + 1e-16
    out1 = p[:, :HID] / den
    h1 = jnp.where(out1 > 0, out1, jnp.exp(out1) - 1.0)
    w2 = w2_ref[...]
    h2 = jnp.dot(h1, w2, preferred_element_type=jnp.float32)
    h2_ref[...] = h2
    xp3 = jnp.dot(h2, w2.T, preferred_element_type=jnp.float32)
    xp3_ref[...] = jnp.concatenate(
        [xp3, jnp.zeros((xp3.shape[0], GW - HID), jnp.float32)], axis=1)
    den_ref[...] = den


def _stage2(acc1p, W2):
    grid = (N // ROW_BLK,)
    h2, xp3, den = pl.pallas_call(
        _stage2_body,
        grid=grid,
        in_specs=[
            pl.BlockSpec((2, ROW_BLK, GW), lambda i: (0, i, 0)),
            pl.BlockSpec((HID, OUT), lambda i: (0, 0)),
        ],
        out_specs=[
            pl.BlockSpec((ROW_BLK, OUT), lambda i: (i, 0)),
            pl.BlockSpec((ROW_BLK, GW), lambda i: (i, 0)),
            pl.BlockSpec((ROW_BLK, 1), lambda i: (i, 0)),
        ],
        out_shape=[
            jax.ShapeDtypeStruct((N, OUT), jnp.float32),
            jax.ShapeDtypeStruct((N, GW), jnp.float32),
            jax.ShapeDtypeStruct((N, 1), jnp.float32),
        ],
    )(acc1p, W2)
    return h2, xp3, den


def _stage3_body(accp_ref, den_ref, w1_ref, h4_ref):
    p = accp_ref[0] + accp_ref[1]
    out3 = p[:, :HID] / den_ref[...]
    h3 = jnp.where(out3 > 0, out3, jnp.exp(out3) - 1.0)
    h4_ref[...] = jnp.dot(h3, w1_ref[...].T, preferred_element_type=jnp.float32)


def _stage3(acc3p, den, W1):
    grid = (N // ROW_BLK,)
    h4 = pl.pallas_call(
        _stage3_body,
        grid=grid,
        in_specs=[
            pl.BlockSpec((2, ROW_BLK, GW), lambda i: (0, i, 0)),
            pl.BlockSpec((ROW_BLK, 1), lambda i: (i, 0)),
            pl.BlockSpec((IN_DIM, HID), lambda i: (0, 0)),
        ],
        out_specs=pl.BlockSpec((ROW_BLK, IN_DIM), lambda i: (i, 0)),
        out_shape=jax.ShapeDtypeStruct((N, IN_DIM), jnp.float32),
    )(acc3p, den, W1)
    return h4


_SC_MESH = plsc.VectorSubcoreMesh(core_axis_name="c", subcore_axis_name="s")


def _zero_rows(rows):
    def zloop(i, _):
        z = jnp.zeros((16,), jnp.float32)
        for c in range(GW // 16):
            rows[i, pl.ds(c * 16, 16)] = z
        return 0
    lax.fori_loop(0, CHUNK, zloop, 0)


def _zero_acc(zsrc, acc, sid):
    r0 = sid * ROWS_PT
    for off in range(0, ROWS_PT, CHUNK):
        sz = min(CHUNK, ROWS_PT - off)
        pltpu.sync_copy(zsrc.at[pl.ds(0, sz)], acc.at[pl.ds(r0 + off, sz)])


def _copy_out(acc, out_hbm, cid, sid):
    r0 = sid * ROWS_PT
    pltpu.sync_copy(acc.at[pl.ds(r0, ROWS_PT)], out_hbm.at[cid, pl.ds(r0, ROWS_PT)])


def _conv1_edge_body(ed, adF, xpF,
                     w_out, accp_out,
                     adv, srcv, dstv, wv, rows, acc, sem):
    cid = lax.axis_index("c")
    sid = lax.axis_index("s")
    tid = cid * 16 + sid
    ebase = tid * EPT

    _zero_rows(rows)
    _zero_acc(rows, acc, sid)
    pltpu.sync_copy(adF.at[:], adv)
    plsc.subcore_barrier()

    def chunk(k, _):
        eb = ebase + k * CHUNK
        pltpu.sync_copy(ed.at[pl.ds(eb, CHUNK)], srcv)
        pltpu.sync_copy(ed.at[pl.ds(EPAD + eb, CHUNK)], dstv)

        pltpu.async_copy(xpF.at[srcv], rows, sem).wait()

        def logit(i, _):
            e16 = jax.lax.iota(jnp.int32, 16) + i * 16
            av = plsc.load_gather(rows, [e16, jnp.full((16,), HID, jnp.int32)])
            d16 = dstv[pl.ds(i * 16, 16)]
            bv = plsc.load_gather(adv, [d16])
            x = av + bv
            sig = 1.0 / (1.0 + jnp.exp(-x))
            wv[pl.ds(i * 16, 16)] = jnp.exp(sig)
            return 0
        lax.fori_loop(0, CHUNK // 16, logit, 0)
        pltpu.sync_copy(wv, w_out.at[pl.ds(eb, CHUNK)])

        def scale(e, _):
            ws = plsc.load_gather(wv, [jnp.broadcast_to(e, (16,))])
            for c in range(HID // 16):
                rows[e, pl.ds(c * 16, 16)] = rows[e, pl.ds(c * 16, 16)] * ws
            rows[e, pl.ds(HID, 16)] = ws
            return 0
        lax.fori_loop(0, CHUNK, scale, 0)

        pltpu.sync_copy(rows, acc.at[dstv], add=True)
        return 0

    lax.fori_loop(0, NCHUNK, chunk, 0)
    plsc.subcore_barrier()
    _copy_out(acc, accp_out, cid, sid)


def _conv1_edges(ed, a_d_pad, xp1):
    kfn = pl.kernel(
        _conv1_edge_body,
        out_type=[
            jax.ShapeDtypeStruct((EPAD,), jnp.float32),
            jax.ShapeDtypeStruct((2, NPAD, GW), jnp.float32),
        ],
        mesh=_SC_MESH,
        compiler_params=pltpu.CompilerParams(needs_layout_passes=False),
        scratch_types=[
            pltpu.VMEM((NPAD,), jnp.float32),
            pltpu.VMEM((CHUNK,), jnp.int32),
            pltpu.VMEM((CHUNK,), jnp.int32),
            pltpu.VMEM((CHUNK,), jnp.float32),
            pltpu.VMEM((CHUNK, GW), jnp.float32),
            pltpu.VMEM_SHARED((NPAD, GW), jnp.float32),
            pltpu.SemaphoreType.DMA,
        ],
    )
    return kfn(ed, a_d_pad, xp1)


def _conv3_edge_body(ed, wF, xpF,
                     accp_out,
                     srcv, dstv, wv, rows, acc, sem):
    cid = lax.axis_index("c")
    sid = lax.axis_index("s")
    tid = cid * 16 + sid
    ebase = tid * EPT

    _zero_rows(rows)
    _zero_acc(rows, acc, sid)
    plsc.subcore_barrier()

    def chunk(k, _):
        eb = ebase + k * CHUNK
        pltpu.sync_copy(ed.at[pl.ds(eb, CHUNK)], srcv)
        pltpu.sync_copy(ed.at[pl.ds(EPAD + eb, CHUNK)], dstv)
        pltpu.sync_copy(wF.at[pl.ds(eb, CHUNK)], wv)

        pltpu.async_copy(xpF.at[srcv], rows, sem).wait()

        def scale(e, _):
            ws = plsc.load_gather(wv, [jnp.broadcast_to(e, (16,))])
            for c in range(HID // 16):
                rows[e, pl.ds(c * 16, 16)] = rows[e, pl.ds(c * 16, 16)] * ws
            return 0
        lax.fori_loop(0, CHUNK, scale, 0)

        pltpu.sync_copy(rows, acc.at[dstv], add=True)
        return 0

    lax.fori_loop(0, NCHUNK, chunk, 0)
    plsc.subcore_barrier()
    _copy_out(acc, accp_out, cid, sid)


def _conv3_edges(ed, w, xp3):
    kfn = pl.kernel(
        _conv3_edge_body,
        out_type=jax.ShapeDtypeStruct((2, NPAD, GW), jnp.float32),
        mesh=_SC_MESH,
        compiler_params=pltpu.CompilerParams(needs_layout_passes=False),
        scratch_types=[
            pltpu.VMEM((CHUNK,), jnp.int32),
            pltpu.VMEM((CHUNK,), jnp.int32),
            pltpu.VMEM((CHUNK,), jnp.float32),
            pltpu.VMEM((CHUNK, GW), jnp.float32),
            pltpu.VMEM_SHARED((NPAD, GW), jnp.float32),
            pltpu.SemaphoreType.DMA,
        ],
    )
    return kfn(ed, w, xp3)


@jax.jit
def kernel(features, edge_index, W1, W2, att1_src, att1_dst):
    src = edge_index[0]
    dst = edge_index[1]
    srcF = jnp.pad(src, (0, EPAD - E))
    dstF = jnp.pad(dst, (0, EPAD - E), constant_values=NPAD - 1)
    ed = jnp.concatenate([srcF, dstF])

    xp1, a_d = _stage1(features, W1, att1_src, att1_dst)
    a_d_pad = jnp.pad(a_d, (0, NPAD - N))

    w, acc1p = _conv1_edges(ed, a_d_pad, xp1)
    h2, xp3, den = _stage2(acc1p[:, :N, :], W2)
    acc3p = _conv3_edges(ed, w, xp3)
    h4 = _stage3(acc3p[:, :N, :], den, W1)
    return (h2, h4)
